# baseline (speedup 1.0000x reference)
import functools

import jax
import jax.numpy as jnp
from jax import lax
from jax.experimental import pallas as pl
from jax.experimental.pallas import tpu as pltpu
from jax.experimental.pallas import tpu_sc as plsc


EMBED = 64
PE_LEN = 256
LANES = 16


def _fuse_tables(token_table, language_table, bounds):
    V, E = token_table.shape
    BLK = 25000
    grid = V // BLK

    def body(b_ref, lang_ref, tok_ref, out_ref):
        i = pl.program_id(0)
        rows = tok_ref[...]
        rid = lax.broadcasted_iota(jnp.int32, (BLK, 1), 0) + i * BLK
        lang = lang_ref[...]
        for off, l in ((0, 3), (2, 2), (4, 1)):
            lo = b_ref[off]
            hi = b_ref[off + 1]
            m = (rid >= lo) & (rid <= hi)
            rows = rows + jnp.where(m, lang[l][None, :], 0.0)
        out_ref[...] = rows

    return pl.pallas_call(
        body,
        grid=(grid,),
        in_specs=[
            pl.BlockSpec(memory_space=pltpu.SMEM),
            pl.BlockSpec((4, E), lambda i: (0, 0)),
            pl.BlockSpec((BLK, E), lambda i: (i, 0)),
        ],
        out_specs=pl.BlockSpec((BLK, E), lambda i: (i, 0)),
        out_shape=jax.ShapeDtypeStruct((V, E), jnp.float32),
    )(bounds, language_table, token_table)


def _sc_gather(fused, seq2d, pe):
    info = plsc.get_sparse_core_info()
    NC, NS = info.num_cores, info.num_subcores
    NW = NC * NS
    TOTAL = seq2d.shape[0] * seq2d.shape[1]
    PER_W = TOTAL // NW
    CHUNK = 512
    IDXW = 128
    NSUB = CHUNK // IDXW
    NCH = PER_W // CHUNK
    mesh = plsc.VectorSubcoreMesh(core_axis_name="c", subcore_axis_name="s")

    @functools.partial(
        pl.kernel,
        mesh=mesh,
        out_type=jax.ShapeDtypeStruct((TOTAL // 2, 2 * EMBED), jnp.float32),
        compiler_params=pltpu.CompilerParams(use_tc_tiling_on_sc=False),
        scratch_types=[
            pltpu.VMEM((NSUB, IDXW), jnp.int32),
            pltpu.VMEM((NSUB, IDXW), jnp.int32),
            pltpu.VMEM((CHUNK, EMBED), jnp.float32),
            pltpu.VMEM((CHUNK, EMBED), jnp.float32),
            pltpu.VMEM((CHUNK // 2, 2 * EMBED), jnp.float32),
            pltpu.VMEM((PE_LEN, EMBED), jnp.float32),
            pltpu.SemaphoreType.DMA,
            pltpu.SemaphoreType.DMA,
        ],
    )
    def k(fused_hbm, seq_hbm, pe_hbm, out_hbm, idx0, idx1, rows0, rows1, pack_v, pe_v, sem0, sem1):
        wid = lax.axis_index("s") * NC + lax.axis_index("c")
        pltpu.sync_copy(pe_hbm, pe_v)

        def issue(ci, idx_v, rows_v, sem):
            brow = wid * (PER_W // IDXW) + ci * NSUB
            pltpu.sync_copy(seq_hbm.at[pl.ds(brow, NSUB)], idx_v)
            for j in range(NSUB):
                pltpu.async_copy(
                    fused_hbm.at[idx_v.at[j]],
                    rows_v.at[pl.ds(j * IDXW, IDXW)],
                    sem,
                )

        def drain(rows_v, sem):
            pltpu.make_async_copy(fused_hbm.at[pl.ds(0, CHUNK)], rows_v, sem).wait()

        def process(ci, rows_v):
            @plsc.parallel_loop(0, CHUNK // 2, unroll=8)
            def pe_body(q):
                for half in range(2):
                    r = 2 * q + half
                    p = r & (PE_LEN - 1)
                    for c in range(EMBED // LANES):
                        src = pl.ds(c * LANES, LANES)
                        dst = pl.ds(half * EMBED + c * LANES, LANES)
                        pack_v[q, dst] = rows_v[r, src] + pe_v[p, src]
            base2 = (wid * PER_W + ci * CHUNK) // 2
            pltpu.sync_copy(pack_v, out_hbm.at[pl.ds(base2, CHUNK // 2)])

        issue(0, idx0, rows0, sem0)

        def pair_body(i2, carry):
            c0 = i2 * 2
            issue(c0 + 1, idx1, rows1, sem1)
            drain(rows0, sem0)
            process(c0, rows0)
            c2 = jnp.minimum(c0 + 2, NCH - 1)
            issue(c2, idx0, rows0, sem0)
            drain(rows1, sem1)
            process(c0 + 1, rows1)
            return carry

        lax.fori_loop(0, NCH // 2, pair_body, 0)
        drain(rows0, sem0)

    return k(fused, seq2d, pe)


def kernel(sequence, token_table, language_table, pe, ms_vocab, eng_vocab, chi_vocab):
    B, S = sequence.shape
    bounds = jnp.stack(
        [
            ms_vocab[0].astype(jnp.int32),
            ms_vocab[-1].astype(jnp.int32),
            eng_vocab[0].astype(jnp.int32),
            eng_vocab[-1].astype(jnp.int32),
            chi_vocab[0].astype(jnp.int32),
            chi_vocab[-1].astype(jnp.int32),
        ]
    )
    fused = _fuse_tables(token_table, language_table, bounds)
    seq2d = sequence.astype(jnp.int32).reshape(-1, 128)
    out = _sc_gather(fused, seq2d, pe)
    return out.reshape(B, S, EMBED)


# speedup vs baseline: 1.0139x; 1.0139x over previous
import functools

import jax
import jax.numpy as jnp
from jax import lax
from jax.experimental import pallas as pl
from jax.experimental.pallas import tpu as pltpu
from jax.experimental.pallas import tpu_sc as plsc


EMBED = 64
PE_LEN = 256
LANES = 16
WIDE = 2 * EMBED


def _fuse_tables(token_table, language_table, bounds):
    V, E = token_table.shape
    BLK = 25000
    grid = V // BLK

    def body(b_ref, lang_ref, tok_ref, out_ref):
        i = pl.program_id(0)
        rows = tok_ref[...]
        rid = lax.broadcasted_iota(jnp.int32, (BLK, 1), 0) + i * BLK
        lang = lang_ref[...]
        for off, l in ((0, 3), (2, 2), (4, 1)):
            lo = b_ref[off]
            hi = b_ref[off + 1]
            m = (rid >= lo) & (rid <= hi)
            rows = rows + jnp.where(m, lang[l][None, :], 0.0)
        out_ref[...] = jnp.concatenate([rows, jnp.zeros_like(rows)], axis=1)

    return pl.pallas_call(
        body,
        grid=(grid,),
        in_specs=[
            pl.BlockSpec(memory_space=pltpu.SMEM),
            pl.BlockSpec((4, E), lambda i: (0, 0)),
            pl.BlockSpec((BLK, E), lambda i: (i, 0)),
        ],
        out_specs=pl.BlockSpec((BLK, WIDE), lambda i: (i, 0)),
        out_shape=jax.ShapeDtypeStruct((V, WIDE), jnp.float32),
    )(bounds, language_table, token_table)


def _sc_gather(fused, seq2d, pe2):
    info = plsc.get_sparse_core_info()
    NC, NS = info.num_cores, info.num_subcores
    NW = NC * NS
    TOTAL = seq2d.shape[0] * seq2d.shape[1]
    PER_W = TOTAL // NW
    CHUNK = 256
    IDXW = 128
    NSUB = CHUNK // IDXW
    GRP = 4
    NG = PER_W // (CHUNK * GRP)
    mesh = plsc.VectorSubcoreMesh(core_axis_name="c", subcore_axis_name="s")

    @functools.partial(
        pl.kernel,
        mesh=mesh,
        out_type=jax.ShapeDtypeStruct((TOTAL // 2, WIDE), jnp.float32),
        compiler_params=pltpu.CompilerParams(use_tc_tiling_on_sc=True),
        scratch_types=[
            pltpu.VMEM((GRP * NSUB, IDXW), jnp.int32),
            pltpu.VMEM((GRP * NSUB, IDXW), jnp.int32),
            pltpu.VMEM((CHUNK, WIDE), jnp.float32),
            pltpu.VMEM((CHUNK, WIDE), jnp.float32),
            pltpu.VMEM((CHUNK // 2, WIDE), jnp.float32),
            pltpu.VMEM((PE_LEN // 2, WIDE), jnp.float32),
            pltpu.SemaphoreType.DMA,
            pltpu.SemaphoreType.DMA,
        ],
    )
    def k(fused_hbm, seq_hbm, pe_hbm, out_hbm, idxA, idxB, rows0, rows1, pack_v, pe_v, sem0, sem1):
        wid = lax.axis_index("s") * NC + lax.axis_index("c")
        pltpu.sync_copy(pe_hbm, pe_v)
        rows = (rows0, rows1)
        sems = (sem0, sem1)

        def fetch_idx(g, idx_v):
            brow = wid * (PER_W // IDXW) + g * GRP * NSUB
            pltpu.sync_copy(seq_hbm.at[pl.ds(brow, GRP * NSUB)], idx_v)

        def issue(idx_v, cl, rows_v, sem):
            for j in range(NSUB):
                pltpu.async_copy(
                    fused_hbm.at[idx_v.at[cl * NSUB + j]],
                    rows_v.at[pl.ds(j * IDXW, IDXW)],
                    sem,
                )

        def drain(rows_v, sem):
            pltpu.make_async_copy(fused_hbm.at[pl.ds(0, CHUNK)], rows_v, sem).wait()

        def process(ci, rows_v):
            @plsc.parallel_loop(0, CHUNK // 2, unroll=8)
            def pe_body(q):
                for half in range(2):
                    r = 2 * q + half
                    for c in range(EMBED // LANES):
                        src = pl.ds(c * LANES, LANES)
                        dst = pl.ds(half * EMBED + c * LANES, LANES)
                        pack_v[q, dst] = rows_v[r, src] + pe_v[q, dst]
            base2 = wid * (PER_W // 2) + ci * (CHUNK // 2)
            pltpu.sync_copy(pack_v, out_hbm.at[pl.ds(base2, CHUNK // 2)])

        fetch_idx(0, idxA)
        issue(idxA, 0, rows0, sem0)

        def super_body(m, carry):
            fetch_idx(2 * m + 1, idxB)
            for pair, idx_cur, idx_nxt in ((0, idxA, idxB), (1, idxB, idxA)):
                g = 2 * m + pair
                for cl in range(GRP):
                    ci = g * GRP + cl
                    cur, nxt = cl % 2, (cl + 1) % 2
                    if cl < GRP - 1:
                        issue(idx_cur, cl + 1, rows[nxt], sems[nxt])
                    else:
                        if pair == 1:
                            fetch_idx(jnp.minimum(2 * m + 2, NG - 1), idxA)
                        issue(idx_nxt, 0, rows[nxt], sems[nxt])
                    drain(rows[cur], sems[cur])
                    process(ci, rows[cur])
            return carry

        lax.fori_loop(0, NG // 2, super_body, 0)
        drain(rows0, sems[0])

    return k(fused, seq2d, pe2)


def kernel(sequence, token_table, language_table, pe, ms_vocab, eng_vocab, chi_vocab):
    B, S = sequence.shape
    bounds = jnp.stack(
        [
            ms_vocab[0].astype(jnp.int32),
            ms_vocab[-1].astype(jnp.int32),
            eng_vocab[0].astype(jnp.int32),
            eng_vocab[-1].astype(jnp.int32),
            chi_vocab[0].astype(jnp.int32),
            chi_vocab[-1].astype(jnp.int32),
        ]
    )
    fused = _fuse_tables(token_table, language_table, bounds)
    seq2d = sequence.astype(jnp.int32).reshape(-1, 128)
    pe2 = pe.reshape(PE_LEN // 2, WIDE)
    out = _sc_gather(fused, seq2d, pe2)
    return out.reshape(B, S, EMBED)
